# baseline (speedup 1.0000x reference)
import jax
import jax.numpy as jnp
from jax import lax
from jax.experimental import pallas as pl
from jax.experimental.pallas import tpu as pltpu
from jax.experimental.pallas import tpu_sc as plsc

N_NODES = 10000
N_PAD = 10240
D_FEAT = 128
F = 16
N_EDGES = 320000
NC, NS = 2, 16
NW = NC * NS
B = 128
EP = 10240
C = EP // B
E_PAD = NW * EP
NB = 4
NH = NB // 2
EPS = (C + NB) * B
ROWS = N_PAD // NS


def _sc_propagate_body(g_hbm, src_hbm, dst_hbm, zero_hbm, out_hbm,
                       src_v, dst_v, buf_v, gtab, acc, *sems):
    gsem = sems[:NB]
    ssem = sems[NB:]
    c = lax.axis_index("c")
    s = lax.axis_index("s")
    w = c * NS + s
    pltpu.sync_copy(zero_hbm, acc.at[pl.ds(s * ROWS, ROWS)])
    pltpu.sync_copy(g_hbm.at[pl.ds(s * ROWS, ROWS)],
                    gtab.at[pl.ds(s * ROWS, ROWS)])
    pltpu.sync_copy(src_hbm.at[w], src_v)
    pltpu.sync_copy(dst_hbm.at[w], dst_v)
    plsc.subcore_barrier()

    bufs = [buf_v.at[b] for b in range(NB)]

    def g_wait(b):
        pltpu.make_async_copy(gtab.at[src_v.at[pl.ds(0, B)]],
                              bufs[b], gsem[b]).wait()

    def s_wait(b):
        pltpu.make_async_copy(bufs[b], acc.at[dst_v.at[0]],
                              ssem[b]).wait()

    def g_issue(j, b):
        off = pl.multiple_of(j * B, B)
        pltpu.async_copy(gtab.at[src_v.at[pl.ds(off, B)]],
                         bufs[b], gsem[b])

    for b in range(NH):
        g_issue(b, b)
    for b in range(NH, NB):
        pltpu.sync_copy(zero_hbm.at[pl.ds(0, B)], bufs[b])
        pltpu.async_copy(bufs[b], acc.at[dst_v.at[0]], ssem[b], add=True)

    def group(jq, carry):
        j0 = jq * NB
        for b in range(NB):
            g_wait(b)
            pltpu.async_copy(bufs[b], acc.at[dst_v.at[j0 + b]],
                             ssem[b], add=True)
            b2 = (b + NH) % NB
            s_wait(b2)
            g_issue(j0 + b + NH, b2)
        return carry

    lax.fori_loop(0, C // NB, group, 0)
    for b in range(NH):
        g_wait(b)
    for b in range(NH, NB):
        s_wait(b)
    plsc.subcore_barrier()
    pltpu.sync_copy(acc.at[pl.ds(s * ROWS, ROWS)],
                    out_hbm.at[c, pl.ds(s * ROWS, ROWS)])


_sc_propagate = pl.kernel(
    _sc_propagate_body,
    out_type=jax.ShapeDtypeStruct((NC, N_PAD, F), jnp.float32),
    mesh=plsc.VectorSubcoreMesh(core_axis_name="c", subcore_axis_name="s"),
    compiler_params=pltpu.CompilerParams(use_tc_tiling_on_sc=False),
    scratch_types=[
        pltpu.VMEM((EPS,), jnp.int32),
        pltpu.VMEM((C, B), jnp.int32),
        pltpu.VMEM((NB, B, F), jnp.float32),
        pltpu.VMEM_SHARED((N_PAD, F), jnp.float32),
        pltpu.VMEM_SHARED((N_PAD, F), jnp.float32),
    ] + [pltpu.SemaphoreType.DMA] * (2 * NB),
)

SK = 8


def _sc_degree_body(ones_hbm, dst_hbm, zero_hbm, out_hbm,
                    dst_v, buf_v, acc, ssem):
    c = lax.axis_index("c")
    s = lax.axis_index("s")
    w = c * NS + s
    pltpu.sync_copy(zero_hbm, acc.at[pl.ds(s * ROWS, ROWS)])
    pltpu.sync_copy(dst_hbm.at[w], dst_v)
    pltpu.sync_copy(ones_hbm, buf_v)
    plsc.subcore_barrier()

    def group(jq, carry):
        j0 = jq * SK
        for b in range(SK):
            pltpu.async_copy(buf_v, acc.at[dst_v.at[j0 + b]], ssem, add=True)
        for b in range(SK):
            pltpu.make_async_copy(buf_v, acc.at[dst_v.at[j0]], ssem).wait()
        return carry

    lax.fori_loop(0, C // SK, group, 0)
    plsc.subcore_barrier()
    pltpu.sync_copy(acc.at[pl.ds(s * ROWS, ROWS)],
                    out_hbm.at[c, pl.ds(s * ROWS, ROWS)])


_sc_degree = pl.kernel(
    _sc_degree_body,
    out_type=jax.ShapeDtypeStruct((NC, N_PAD, F), jnp.float32),
    mesh=plsc.VectorSubcoreMesh(core_axis_name="c", subcore_axis_name="s"),
    compiler_params=pltpu.CompilerParams(use_tc_tiling_on_sc=False),
    scratch_types=[
        pltpu.VMEM((C, B), jnp.int32),
        pltpu.VMEM((B, F), jnp.float32),
        pltpu.VMEM_SHARED((N_PAD, F), jnp.float32),
        pltpu.SemaphoreType.DMA,
    ],
)


def _prep_body(deg0_ref, deg1_ref, x_ref, w1_ref, dinv_ref, g1_ref):
    dinv = lax.rsqrt(deg0_ref[...] + deg1_ref[...] + 1.0)
    dinv_ref[...] = dinv
    h = jnp.dot(x_ref[...], w1_ref[...], preferred_element_type=jnp.float32)
    g1_ref[...] = dinv * h


_prep = pl.pallas_call(
    _prep_body,
    out_shape=(jax.ShapeDtypeStruct((N_PAD, F), jnp.float32),
               jax.ShapeDtypeStruct((N_PAD, F), jnp.float32)),
)


def _mid_body(s0_ref, s1_ref, g_ref, dinv_ref, b_ref, w_ref, gout_ref):
    dinv = dinv_ref[...]
    a = jnp.maximum(dinv * (s0_ref[...] + s1_ref[...] + g_ref[...]) + b_ref[...], 0.0)
    gout_ref[...] = dinv * jnp.dot(a, w_ref[...], preferred_element_type=jnp.float32)


_mid = pl.pallas_call(
    _mid_body,
    out_shape=jax.ShapeDtypeStruct((N_PAD, F), jnp.float32),
)


def _final_body(s0_ref, s1_ref, g_ref, dinv_ref, b_ref, out_ref):
    z = dinv_ref[...] * (s0_ref[...] + s1_ref[...] + g_ref[...]) + b_ref[...]
    mask = lax.broadcasted_iota(jnp.int32, (N_PAD, F), 1) < 6
    zm = jnp.where(mask, z, -1e30)
    zmax = jnp.max(zm, axis=1, keepdims=True)
    e = jnp.where(mask, jnp.exp(zm - zmax), 0.0)
    p = e / jnp.sum(e, axis=1, keepdims=True)
    ep = jnp.where(mask, jnp.exp(p), 0.0)
    out_ref[...] = p - jnp.log(jnp.sum(ep, axis=1, keepdims=True))


_final = pl.pallas_call(
    _final_body,
    out_shape=jax.ShapeDtypeStruct((N_PAD, F), jnp.float32),
)


def kernel(x, edge_index, W1, b1, W2, b2, W3, b3):
    ei = edge_index.astype(jnp.int32)
    pad = jnp.full((E_PAD - N_EDGES,), N_NODES, jnp.int32)
    src_t = jnp.concatenate([ei[0], pad]).reshape(NW, EP)
    src_t = jnp.pad(src_t, ((0, 0), (0, EPS - EP)),
                    constant_values=N_NODES)
    dst_t = jnp.concatenate([ei[1], pad]).reshape(NW, C, B)
    xp = jnp.pad(x, ((0, N_PAD - N_NODES), (0, 0)))
    w3p = jnp.pad(W3, ((0, 0), (0, F - W3.shape[1])))
    b1r = b1.reshape(1, F)
    b2r = b2.reshape(1, F)
    b3r = jnp.pad(b3, (0, F - b3.shape[0])).reshape(1, F)
    zeros = jnp.zeros((ROWS, F), jnp.float32)
    ones = jnp.ones((B, F), jnp.float32)

    deg = _sc_degree(ones, dst_t, zeros)
    dinv, g1 = _prep(deg[0], deg[1], xp, W1)
    s1 = _sc_propagate(g1, src_t, dst_t, zeros)
    g2 = _mid(s1[0], s1[1], g1, dinv, b1r, W2)
    s2 = _sc_propagate(g2, src_t, dst_t, zeros)
    g3 = _mid(s2[0], s2[1], g2, dinv, b2r, w3p)
    s3 = _sc_propagate(g3, src_t, dst_t, zeros)
    out = _final(s3[0], s3[1], g3, dinv, b3r)
    return out[:N_NODES, :6]


# speedup vs baseline: 1.0292x; 1.0292x over previous
import jax
import jax.numpy as jnp
from jax import lax
from jax.experimental import pallas as pl
from jax.experimental.pallas import tpu as pltpu
from jax.experimental.pallas import tpu_sc as plsc

N_NODES = 10000
N_PAD = 10240
D_FEAT = 128
F = 16
F3 = 8
N_EDGES = 320000
NC, NS = 2, 16
NW = NC * NS
B = 128
EP = 10240
C = EP // B
E_PAD = NW * EP
EPS = (C + 2) * B
ROWS = N_PAD // NS


def _make_sc_propagate(f):
    def body(g_hbm, src_hbm, dst_hbm, zero_hbm, out_hbm,
             src_v, dst_v, buf_v, gtab, acc, gsemA, gsemB):
        c = lax.axis_index("c")
        s = lax.axis_index("s")
        w = c * NS + s
        pltpu.sync_copy(zero_hbm, acc.at[pl.ds(s * ROWS, ROWS)])
        pltpu.sync_copy(g_hbm.at[pl.ds(s * ROWS, ROWS)],
                        gtab.at[pl.ds(s * ROWS, ROWS)])
        pltpu.sync_copy(src_hbm.at[w], src_v)
        pltpu.sync_copy(dst_hbm.at[w], dst_v)
        plsc.subcore_barrier()

        bufA = buf_v.at[0]
        bufB = buf_v.at[1]
        pltpu.async_copy(gtab.at[src_v.at[pl.ds(0, B)]], bufA, gsemA)
        pltpu.async_copy(gtab.at[src_v.at[pl.ds(B, B)]], bufB, gsemB)

        def pair(jp, carry):
            j0 = jp * 2
            offA = pl.multiple_of((j0 + 2) * B, B)
            offB = pl.multiple_of((j0 + 3) * B, B)
            pltpu.make_async_copy(gtab.at[src_v.at[pl.ds(0, B)]],
                                  bufA, gsemA).wait()
            pltpu.sync_copy(bufA, acc.at[dst_v.at[j0]], add=True)
            pltpu.async_copy(gtab.at[src_v.at[pl.ds(offA, B)]], bufA, gsemA)
            pltpu.make_async_copy(gtab.at[src_v.at[pl.ds(0, B)]],
                                  bufB, gsemB).wait()
            pltpu.sync_copy(bufB, acc.at[dst_v.at[j0 + 1]], add=True)
            pltpu.async_copy(gtab.at[src_v.at[pl.ds(offB, B)]], bufB, gsemB)
            return carry

        lax.fori_loop(0, C // 2, pair, 0)
        pltpu.make_async_copy(gtab.at[src_v.at[pl.ds(0, B)]],
                              bufA, gsemA).wait()
        pltpu.make_async_copy(gtab.at[src_v.at[pl.ds(0, B)]],
                              bufB, gsemB).wait()
        plsc.subcore_barrier()
        pltpu.sync_copy(acc.at[pl.ds(s * ROWS, ROWS)],
                        out_hbm.at[c, pl.ds(s * ROWS, ROWS)])

    return pl.kernel(
        body,
        out_type=jax.ShapeDtypeStruct((NC, N_PAD, f), jnp.float32),
        mesh=plsc.VectorSubcoreMesh(core_axis_name="c", subcore_axis_name="s"),
        compiler_params=pltpu.CompilerParams(use_tc_tiling_on_sc=False),
        scratch_types=[
            pltpu.VMEM((EPS,), jnp.int32),
            pltpu.VMEM((C, B), jnp.int32),
            pltpu.VMEM((2, B, f), jnp.float32),
            pltpu.VMEM_SHARED((N_PAD, f), jnp.float32),
            pltpu.VMEM_SHARED((N_PAD, f), jnp.float32),
            pltpu.SemaphoreType.DMA,
            pltpu.SemaphoreType.DMA,
        ],
    )


_sc_propagate16 = _make_sc_propagate(F)
_sc_propagate8 = _make_sc_propagate(F3)

SK = 8


def _sc_degree_body(ones_hbm, dst_hbm, zero_hbm, out_hbm,
                    dst_v, buf_v, acc, ssem):
    c = lax.axis_index("c")
    s = lax.axis_index("s")
    w = c * NS + s
    pltpu.sync_copy(zero_hbm, acc.at[pl.ds(s * ROWS, ROWS)])
    pltpu.sync_copy(dst_hbm.at[w], dst_v)
    pltpu.sync_copy(ones_hbm, buf_v)
    plsc.subcore_barrier()

    def group(jq, carry):
        j0 = jq * SK
        for b in range(SK):
            pltpu.async_copy(buf_v, acc.at[dst_v.at[j0 + b]], ssem, add=True)
        for b in range(SK):
            pltpu.make_async_copy(buf_v, acc.at[dst_v.at[j0]], ssem).wait()
        return carry

    lax.fori_loop(0, C // SK, group, 0)
    plsc.subcore_barrier()
    pltpu.sync_copy(acc.at[pl.ds(s * ROWS, ROWS)],
                    out_hbm.at[c, pl.ds(s * ROWS, ROWS)])


_sc_degree = pl.kernel(
    _sc_degree_body,
    out_type=jax.ShapeDtypeStruct((NC, N_PAD, F), jnp.float32),
    mesh=plsc.VectorSubcoreMesh(core_axis_name="c", subcore_axis_name="s"),
    compiler_params=pltpu.CompilerParams(use_tc_tiling_on_sc=False),
    scratch_types=[
        pltpu.VMEM((C, B), jnp.int32),
        pltpu.VMEM((B, F), jnp.float32),
        pltpu.VMEM_SHARED((N_PAD, F), jnp.float32),
        pltpu.SemaphoreType.DMA,
    ],
)


def _prep_body(deg0_ref, deg1_ref, x_ref, w1_ref, dinv_ref, g1_ref):
    dinv = lax.rsqrt(deg0_ref[...] + deg1_ref[...] + 1.0)
    dinv_ref[...] = dinv
    h = jnp.dot(x_ref[...], w1_ref[...], preferred_element_type=jnp.float32)
    g1_ref[...] = dinv * h


_prep = pl.pallas_call(
    _prep_body,
    out_shape=(jax.ShapeDtypeStruct((N_PAD, F), jnp.float32),
               jax.ShapeDtypeStruct((N_PAD, F), jnp.float32)),
)


def _mid1_body(s0_ref, s1_ref, g_ref, dinv_ref, b_ref, w_ref, gout_ref):
    dinv = dinv_ref[...]
    a = jnp.maximum(dinv * (s0_ref[...] + s1_ref[...] + g_ref[...])
                    + b_ref[...], 0.0)
    gout_ref[...] = dinv * jnp.dot(a, w_ref[...],
                                   preferred_element_type=jnp.float32)


_mid1 = pl.pallas_call(
    _mid1_body,
    out_shape=jax.ShapeDtypeStruct((N_PAD, F), jnp.float32),
)


def _mid2_body(s0_ref, s1_ref, g_ref, dinv_ref, b_ref, w_ref, gout_ref):
    dinv = dinv_ref[...]
    a = jnp.maximum(dinv * (s0_ref[...] + s1_ref[...] + g_ref[...])
                    + b_ref[...], 0.0)
    gout_ref[...] = dinv[:, :F3] * jnp.dot(a, w_ref[...],
                                           preferred_element_type=jnp.float32)


_mid2 = pl.pallas_call(
    _mid2_body,
    out_shape=jax.ShapeDtypeStruct((N_PAD, F3), jnp.float32),
)


def _final_body(s0_ref, s1_ref, g_ref, dinv_ref, b_ref, out_ref):
    z = (dinv_ref[...][:, :F3] * (s0_ref[...] + s1_ref[...] + g_ref[...])
         + b_ref[...])
    mask = lax.broadcasted_iota(jnp.int32, (N_PAD, F3), 1) < 6
    zm = jnp.where(mask, z, -1e30)
    zmax = jnp.max(zm, axis=1, keepdims=True)
    e = jnp.where(mask, jnp.exp(zm - zmax), 0.0)
    p = e / jnp.sum(e, axis=1, keepdims=True)
    ep = jnp.where(mask, jnp.exp(p), 0.0)
    out_ref[...] = p - jnp.log(jnp.sum(ep, axis=1, keepdims=True))


_final = pl.pallas_call(
    _final_body,
    out_shape=jax.ShapeDtypeStruct((N_PAD, F3), jnp.float32),
)


def kernel(x, edge_index, W1, b1, W2, b2, W3, b3):
    ei = edge_index.astype(jnp.int32)
    pad = jnp.full((E_PAD - N_EDGES,), N_NODES, jnp.int32)
    src_t = jnp.concatenate([ei[0], pad]).reshape(NW, EP)
    src_t = jnp.pad(src_t, ((0, 0), (0, EPS - EP)),
                    constant_values=N_NODES)
    dst_t = jnp.concatenate([ei[1], pad]).reshape(NW, C, B)
    xp = jnp.pad(x, ((0, N_PAD - N_NODES), (0, 0)))
    w3p = jnp.pad(W3, ((0, 0), (0, F3 - W3.shape[1])))
    b1r = b1.reshape(1, F)
    b2r = b2.reshape(1, F)
    b3r = jnp.pad(b3, (0, F3 - b3.shape[0])).reshape(1, F3)
    zeros = jnp.zeros((ROWS, F), jnp.float32)
    zeros8 = jnp.zeros((ROWS, F3), jnp.float32)
    ones = jnp.ones((B, F), jnp.float32)

    deg = _sc_degree(ones, dst_t, zeros)
    dinv, g1 = _prep(deg[0], deg[1], xp, W1)
    s1 = _sc_propagate16(g1, src_t, dst_t, zeros)
    g2 = _mid1(s1[0], s1[1], g1, dinv, b1r, W2)
    s2 = _sc_propagate16(g2, src_t, dst_t, zeros)
    g3 = _mid2(s2[0], s2[1], g2, dinv, b2r, w3p)
    s3 = _sc_propagate8(g3, src_t, dst_t, zeros8)
    out = _final(s3[0], s3[1], g3, dinv, b3r)
    return out[:N_NODES, :6]
